# baseline (speedup 1.0000x reference)
import functools

import jax
import jax.numpy as jnp
from jax import lax
from jax.experimental import pallas as pl
from jax.experimental.pallas import tpu as pltpu
from jax.experimental.pallas import tpu_sc as plsc

N = 10000
E = 320000
D = 128
ED = 16
H = 128
G = 64

NC = 2
NS = 16
NW = NC * NS
CH = 40
NCHUNK = E // CH
EPW = E // NW
CPW = EPW // CH
RPT = 624
TAIL = N - NS * RPT

BLK = 1000
NBLK = N // BLK
EBLK = 4000
NEBLK = E // EBLK

f32 = jnp.float32

_MESH = plsc.VectorSubcoreMesh(
    core_axis_name="c", subcore_axis_name="s", num_cores=NC, num_subcores=NS
)


def _zero_fill(buf, rows, cols, value=0.0):
    def body(r, _):
        for j in range(cols // 16):
            buf[r, pl.ds(j * 16, 16)] = jnp.full((16,), value, f32)
        return 0

    lax.fori_loop(0, rows, body, 0)


def _zero_shared(vbuf, shared, sid):
    base = sid * RPT
    for k in range(RPT // CH):
        pltpu.sync_copy(vbuf, shared.at[pl.ds(base + k * CH, CH)])
    rem = RPT % CH
    if rem:
        pltpu.sync_copy(
            vbuf.at[pl.ds(0, rem)], shared.at[pl.ds(base + (RPT // CH) * CH, rem)]
        )

    @pl.when(sid == NS - 1)
    def _():
        pltpu.sync_copy(vbuf.at[pl.ds(0, TAIL)], shared.at[pl.ds(NS * RPT, TAIL)])


def _writeout_shared(shared, out_hbm, cid, sid):
    pltpu.sync_copy(
        shared.at[pl.ds(sid * RPT, RPT)], out_hbm.at[cid, pl.ds(sid * RPT, RPT)]
    )

    @pl.when(sid == NS - 1)
    def _():
        pltpu.sync_copy(
            shared.at[pl.ds(NS * RPT, TAIL)], out_hbm.at[cid, pl.ds(NS * RPT, TAIL)]
        )


def _edge_body(t_hbm, c_hbm, rc_hbm, out_hbm,
               s_shared, ircb, g0, c0, g1, c1,
               sg0, sc0, sg1, sc1, ss0, ss1):
    sets = [(g0, c0, sg0, sc0, ss0), (g1, c1, sg1, sc1, ss1)]
    cid = lax.axis_index("c")
    sid = lax.axis_index("s")
    wid = sid * NC + cid

    _zero_fill(g0, CH, H)
    _zero_shared(g0.at[pl.ds(0, CH)], s_shared, sid)
    pltpu.sync_copy(rc_hbm.at[pl.ds(wid * 2 * EPW, 2 * EPW)], ircb)
    plsc.subcore_barrier()

    def issue(s, t):
        gb, cb, sg, sc, ss = sets[s]
        pltpu.async_copy(t_hbm.at[ircb.at[pl.ds(t * 2 * CH, 2 * CH)]], gb, sg)
        pltpu.async_copy(c_hbm.at[pl.ds(wid * EPW + t * CH, CH)], cb, sc)

    def wait_g(s):
        gb, cb, sg, sc, ss = sets[s]
        pltpu.make_async_copy(t_hbm.at[ircb.at[pl.ds(0, 2 * CH)]], gb, sg).wait()
        pltpu.make_async_copy(c_hbm.at[pl.ds(0, CH)], cb, sc).wait()

    def compute(s):
        gb, cb, sg, sc, ss = sets[s]

        def crow(r, _):
            for j in range(H // 16):
                sl = pl.ds(j * 16, 16)
                v = gb[r, sl] + gb[CH + r, sl] + cb[r, sl]
                gb[r, sl] = jnp.maximum(v, 0.0)
            return 0

        lax.fori_loop(0, CH, crow, 0)

    def scat_start(s, t):
        gb, cb, sg, sc, ss = sets[s]
        pltpu.async_copy(gb.at[pl.ds(0, CH)],
                         s_shared.at[ircb.at[pl.ds(t * 2 * CH, CH)]], ss,
                         add=True)

    def wait_s(s):
        gb, cb, sg, sc, ss = sets[s]
        pltpu.make_async_copy(gb.at[pl.ds(0, CH)],
                              s_shared.at[ircb.at[pl.ds(0, CH)]], ss).wait()

    issue(0, 0)
    wait_g(0)
    issue(1, 1)
    compute(0)
    scat_start(0, 0)

    def pair(i, _):
        t1 = 2 * i + 1
        wait_g(1)
        wait_s(0)
        issue(0, jnp.minimum(t1 + 1, CPW - 1))
        compute(1)
        scat_start(1, t1)
        t2 = t1 + 1
        wait_g(0)
        wait_s(1)
        issue(1, jnp.minimum(t2 + 1, CPW - 1))
        compute(0)
        scat_start(0, t2)
        return 0

    if CPW % 2 == 1:
        lax.fori_loop(0, (CPW - 1) // 2, pair, 0)
        wait_g(1)
        wait_s(0)
    else:
        lax.fori_loop(0, (CPW - 2) // 2, pair, 0)
        wait_g(1)
        compute(1)
        scat_start(1, CPW - 1)
        wait_s(0)
        wait_s(1)

    plsc.subcore_barrier()
    _writeout_shared(s_shared, out_hbm, cid, sid)

_edge_call = pl.kernel(
    _edge_body,
    out_type=jax.ShapeDtypeStruct((NC, N, H), f32),
    mesh=_MESH,
    scratch_types=[
        pltpu.VMEM_SHARED((N, H), f32),
        pltpu.VMEM((2 * EPW,), jnp.int32),
        pltpu.VMEM((2 * CH, H), f32),
        pltpu.VMEM((CH, H), f32),
        pltpu.VMEM((2 * CH, H), f32),
        pltpu.VMEM((CH, H), f32),
    ] + [pltpu.SemaphoreType.DMA] * 6,
)


def _deg_body(row_hbm, out_hbm, d_shared, irb, vbuf, ss0, ss1):
    cid = lax.axis_index("c")
    sid = lax.axis_index("s")
    wid = sid * NC + cid

    _zero_fill(vbuf, CH, ED)
    _zero_shared(vbuf, d_shared, sid)
    _zero_fill(vbuf, CH, ED, value=1.0)
    pltpu.sync_copy(row_hbm.at[pl.ds(wid * EPW, EPW)], irb)
    plsc.subcore_barrier()

    def scat_start(sem, t):
        pltpu.async_copy(vbuf, d_shared.at[irb.at[pl.ds(t * CH, CH)]], sem,
                         add=True)

    def wait_s(sem):
        pltpu.make_async_copy(vbuf, d_shared.at[irb.at[pl.ds(0, CH)]], sem).wait()

    scat_start(ss0, 0)
    scat_start(ss1, 1)

    def chunk(i, _):
        wait_s(ss0)
        scat_start(ss0, 2 * i + 2)
        wait_s(ss1)
        scat_start(ss1, 2 * i + 3)
        return 0

    if CPW % 2 == 1:
        lax.fori_loop(0, (CPW - 3) // 2, chunk, 0)
        wait_s(ss0)
        scat_start(ss0, CPW - 1)
        wait_s(ss1)
        wait_s(ss0)
    else:
        lax.fori_loop(0, (CPW - 2) // 2, chunk, 0)
        wait_s(ss0)
        wait_s(ss1)
    plsc.subcore_barrier()
    _writeout_shared(d_shared, out_hbm, cid, sid)


_deg_call = pl.kernel(
    _deg_body,
    out_type=jax.ShapeDtypeStruct((NC, N, ED), f32),
    mesh=_MESH,
    scratch_types=[
        pltpu.VMEM_SHARED((N, ED), f32),
        pltpu.VMEM((EPW,), jnp.int32),
        pltpu.VMEM((CH, ED), f32),
        pltpu.SemaphoreType.DMA,
        pltpu.SemaphoreType.DMA,
    ],
)


def _full(shape):
    return pl.BlockSpec(shape, lambda i: (0,) * len(shape))


def _emb_ab_body(x_ref, embWT, embb, w1aT, w1bT, h_ref, a_ref, b_ref):
    h = jnp.dot(x_ref[...], embWT[...], preferred_element_type=f32) + embb[...]
    h_ref[...] = h
    a_ref[...] = jnp.dot(h, w1aT[...], preferred_element_type=f32)
    b_ref[...] = jnp.dot(h, w1bT[...], preferred_element_type=f32)


_emb_ab_call = pl.pallas_call(
    _emb_ab_body,
    grid=(NBLK,),
    in_specs=[
        pl.BlockSpec((BLK, D), lambda i: (i, 0)),
        _full((D, H)), _full((1, H)), _full((H, H)), _full((H, H)),
    ],
    out_specs=[pl.BlockSpec((BLK, H), lambda i: (i, 0))] * 3,
    out_shape=[jax.ShapeDtypeStruct((N, H), f32)] * 3,
)


def _edgec_body(ea_ref, w0, b0, w1, b1, c0_ref, c1_ref):
    ea = ea_ref[...]
    c0_ref[...] = jnp.dot(ea, w0[...], preferred_element_type=f32) + b0[...]
    c1_ref[...] = jnp.dot(ea, w1[...], preferred_element_type=f32) + b1[...]


_edgec_call = pl.pallas_call(
    _edgec_body,
    grid=(NEBLK,),
    in_specs=[
        pl.BlockSpec((EBLK, ED), lambda i: (i, 0)),
        _full((ED, H)), _full((1, H)), _full((ED, H)), _full((1, H)),
    ],
    out_specs=[pl.BlockSpec((EBLK, H), lambda i: (i, 0))] * 2,
    out_shape=[jax.ShapeDtypeStruct((E, H), f32)] * 2,
)


def _gru_core(s2, deg2, h, w2T, b2, wihT, whhT, bih, bhh):
    s = s2[0] + s2[1]
    degc = deg2[0, :, :1] + deg2[1, :, :1]
    agg = jnp.dot(s, w2T[...], preferred_element_type=f32) + degc * b2[...]
    gi = jnp.dot(agg, wihT[...], preferred_element_type=f32) + bih[...]
    gh = jnp.dot(h, whhT[...], preferred_element_type=f32) + bhh[...]
    r = jax.nn.sigmoid(gi[:, :H] + gh[:, :H])
    z = jax.nn.sigmoid(gi[:, H:2 * H] + gh[:, H:2 * H])
    n = jnp.tanh(gi[:, 2 * H:] + r * gh[:, 2 * H:])
    return (1.0 - z) * n + z * h


def _gru_ab_body(s2, deg2, h_ref, w2T, b2, wihT, whhT, bih, bhh, w1aT, w1bT,
                 h1_ref, a_ref, b_ref):
    hn = _gru_core(s2, deg2, h_ref[...], w2T, b2, wihT, whhT, bih, bhh)
    h1_ref[...] = hn
    a_ref[...] = jnp.dot(hn, w1aT[...], preferred_element_type=f32)
    b_ref[...] = jnp.dot(hn, w1bT[...], preferred_element_type=f32)


_gru_ab_call = pl.pallas_call(
    _gru_ab_body,
    grid=(NBLK,),
    in_specs=[
        pl.BlockSpec((NC, BLK, H), lambda i: (0, i, 0)),
        pl.BlockSpec((NC, BLK, ED), lambda i: (0, i, 0)),
        pl.BlockSpec((BLK, H), lambda i: (i, 0)),
        _full((H, H)), _full((1, H)), _full((H, 3 * H)), _full((H, 3 * H)),
        _full((1, 3 * H)), _full((1, 3 * H)), _full((H, H)), _full((H, H)),
    ],
    out_specs=[pl.BlockSpec((BLK, H), lambda i: (i, 0))] * 3,
    out_shape=[jax.ShapeDtypeStruct((N, H), f32)] * 3,
)


def _gru_pool_body(s2, deg2, h_ref, w2T, b2, wihT, whhT, bih, bhh, batch_ref,
                   rw1T, rb1, rw2T, rb2, out_ref, sums, counts):
    i = pl.program_id(0)
    hn = _gru_core(s2, deg2, h_ref[...], w2T, b2, wihT, whhT, bih, bhh)
    bvec = batch_ref[0, 0, :]
    gids = lax.broadcasted_iota(jnp.int32, (G, BLK), 0)
    one_hot = (bvec[None, :] == gids).astype(f32)
    part = jnp.dot(one_hot, hn, preferred_element_type=f32)
    cnt = jnp.broadcast_to(jnp.sum(one_hot, axis=1, keepdims=True), (G, H))

    @pl.when(i == 0)
    def _():
        sums[...] = part
        counts[...] = cnt

    @pl.when(i > 0)
    def _():
        sums[...] += part
        counts[...] += cnt

    @pl.when(i == NBLK - 1)
    def _():
        pooled = sums[...] / jnp.maximum(counts[...], 1.0)
        t = jnp.maximum(
            jnp.dot(pooled, rw1T[...], preferred_element_type=f32) + rb1[...], 0.0
        )
        o = jnp.dot(t, rw2T[...], preferred_element_type=f32) + rb2[...]
        out_ref[...] = jax.nn.sigmoid(o)


_gru_pool_call = pl.pallas_call(
    _gru_pool_body,
    grid=(NBLK,),
    in_specs=[
        pl.BlockSpec((NC, BLK, H), lambda i: (0, i, 0)),
        pl.BlockSpec((NC, BLK, ED), lambda i: (0, i, 0)),
        pl.BlockSpec((BLK, H), lambda i: (i, 0)),
        _full((H, H)), _full((1, H)), _full((H, 3 * H)), _full((H, 3 * H)),
        _full((1, 3 * H)), _full((1, 3 * H)),
        pl.BlockSpec((1, 1, BLK), lambda i: (i, 0, 0)),
        _full((H, H)), _full((1, H)), _full((H, 1)), _full((1, 1)),
    ],
    out_specs=pl.BlockSpec((G, 1), lambda i: (0, 0)),
    out_shape=jax.ShapeDtypeStruct((G, 1), f32),
    scratch_shapes=[pltpu.VMEM((G, H), f32), pltpu.VMEM((G, H), f32)],
)


def kernel(x, edge_index, edge_attr, batch, emb_W, emb_b,
           l0_mW1, l0_mb1, l0_mW2, l0_mb2, l0_Wih, l0_Whh, l0_bih, l0_bhh,
           l1_mW1, l1_mb1, l1_mW2, l1_mb2, l1_Wih, l1_Whh, l1_bih, l1_bhh,
           r_W1, r_b1, r_W2, r_b2):
    row = edge_index[0]
    rc = jnp.concatenate(
        [row.reshape(-1, CH), edge_index[1].reshape(-1, CH) + N], axis=1
    ).reshape(-1)
    batch3 = batch.astype(jnp.int32).reshape(NBLK, 1, BLK)

    def r1(v):
        return v.reshape(1, -1)

    w = {}
    for l, (mW1, mb1, mW2, mb2, Wih, Whh, bih, bhh) in enumerate(
        [(l0_mW1, l0_mb1, l0_mW2, l0_mb2, l0_Wih, l0_Whh, l0_bih, l0_bhh),
         (l1_mW1, l1_mb1, l1_mW2, l1_mb2, l1_Wih, l1_Whh, l1_bih, l1_bhh)]
    ):
        w[l] = dict(
            w1aT=mW1[:, :H].T, w1bT=mW1[:, H:2 * H].T, w1cT=mW1[:, 2 * H:].T,
            mb1=r1(mb1), w2T=mW2.T, mb2=r1(mb2), wihT=Wih.T, whhT=Whh.T,
            bih=r1(bih), bhh=r1(bhh),
        )

    c0, c1 = _edgec_call(edge_attr, w[0]["w1cT"], w[0]["mb1"],
                         w[1]["w1cT"], w[1]["mb1"])
    deg2 = _deg_call(row)
    h0, a0, b0 = _emb_ab_call(x, emb_W.T, r1(emb_b), w[0]["w1aT"], w[0]["w1bT"])
    s0 = _edge_call(jnp.concatenate([a0, b0], axis=0), c0, rc)
    h1, a1, b1 = _gru_ab_call(
        s0, deg2, h0, w[0]["w2T"], w[0]["mb2"], w[0]["wihT"], w[0]["whhT"],
        w[0]["bih"], w[0]["bhh"], w[1]["w1aT"], w[1]["w1bT"]
    )
    s1 = _edge_call(jnp.concatenate([a1, b1], axis=0), c1, rc)
    out = _gru_pool_call(
        s1, deg2, h1, w[1]["w2T"], w[1]["mb2"], w[1]["wihT"], w[1]["whhT"],
        w[1]["bih"], w[1]["bhh"], batch3, r_W1.T, r1(r_b1), r_W2.T,
        r_b2.reshape(1, 1)
    )
    return out


# speedup vs baseline: 1.0283x; 1.0283x over previous
import functools

import jax
import jax.numpy as jnp
import numpy as np
from jax import lax
from jax.experimental import pallas as pl
from jax.experimental.pallas import tpu as pltpu
from jax.experimental.pallas import tpu_sc as plsc

N = 10000
E = 320000
D = 128
ED = 16
H = 128
G = 64

NC = 2
NS = 16
NW = NC * NS
CH = 40
NCHUNK = E // CH
EPW = E // NW
CPW = EPW // CH
RPT = 624
TAIL = N - NS * RPT

BLK = 1000
NBLK = N // BLK
EBLK = 4000
NEBLK = E // EBLK

f32 = jnp.float32

_MESH = plsc.VectorSubcoreMesh(
    core_axis_name="c", subcore_axis_name="s", num_cores=NC, num_subcores=NS
)


def _zero_fill(buf, rows, cols, value=0.0):
    def body(r, _):
        for j in range(cols // 16):
            buf[r, pl.ds(j * 16, 16)] = jnp.full((16,), value, f32)
        return 0

    lax.fori_loop(0, rows, body, 0)


def _zero_shared(vbuf, shared, sid):
    base = sid * RPT
    for k in range(RPT // CH):
        pltpu.sync_copy(vbuf, shared.at[pl.ds(base + k * CH, CH)])
    rem = RPT % CH
    if rem:
        pltpu.sync_copy(
            vbuf.at[pl.ds(0, rem)], shared.at[pl.ds(base + (RPT // CH) * CH, rem)]
        )

    @pl.when(sid == NS - 1)
    def _():
        pltpu.sync_copy(vbuf.at[pl.ds(0, TAIL)], shared.at[pl.ds(NS * RPT, TAIL)])


def _writeout_shared(shared, out_hbm, cid, sid):
    pltpu.sync_copy(
        shared.at[pl.ds(sid * RPT, RPT)], out_hbm.at[cid, pl.ds(sid * RPT, RPT)]
    )

    @pl.when(sid == NS - 1)
    def _():
        pltpu.sync_copy(
            shared.at[pl.ds(NS * RPT, TAIL)], out_hbm.at[cid, pl.ds(NS * RPT, TAIL)]
        )


def _edge_body(a_hbm, b_hbm, c_hbm, row_hbm, col_hbm, out_hbm,
               s_shared, irb, icb, a0, b0, c0, a1, b1, c1,
               sa0, sb0, sc0, sa1, sb1, sc1, ss0, ss1):
    sets = [(a0, b0, c0, sa0, sb0, sc0, ss0), (a1, b1, c1, sa1, sb1, sc1, ss1)]
    cid = lax.axis_index("c")
    sid = lax.axis_index("s")
    wid = sid * NC + cid

    _zero_fill(c0, CH, H)
    _zero_shared(c0, s_shared, sid)
    pltpu.sync_copy(row_hbm.at[pl.ds(wid * EPW, EPW)], irb)
    pltpu.sync_copy(col_hbm.at[pl.ds(wid * EPW, EPW)], icb)
    plsc.subcore_barrier()

    def issue(s, t):
        ab, bb, cb, sa, sb, sc, ss = sets[s]
        pltpu.async_copy(a_hbm.at[irb.at[pl.ds(t * CH, CH)]], ab, sa)
        pltpu.async_copy(b_hbm.at[icb.at[pl.ds(t * CH, CH)]], bb, sb)
        pltpu.async_copy(c_hbm.at[pl.ds(wid * EPW + t * CH, CH)], cb, sc)

    def wait_g(s):
        ab, bb, cb, sa, sb, sc, ss = sets[s]
        pltpu.make_async_copy(a_hbm.at[irb.at[pl.ds(0, CH)]], ab, sa).wait()
        pltpu.make_async_copy(b_hbm.at[icb.at[pl.ds(0, CH)]], bb, sb).wait()
        pltpu.make_async_copy(c_hbm.at[pl.ds(0, CH)], cb, sc).wait()

    def compute(s):
        ab, bb, cb, sa, sb, sc, ss = sets[s]

        def crow(r, _):
            for j in range(H // 16):
                sl = pl.ds(j * 16, 16)
                v = ab[r, sl] + bb[r, sl] + cb[r, sl]
                cb[r, sl] = jnp.maximum(v, 0.0)
            return 0

        lax.fori_loop(0, CH, crow, 0)

    def scat_start(s, t):
        ab, bb, cb, sa, sb, sc, ss = sets[s]
        pltpu.async_copy(cb, s_shared.at[irb.at[pl.ds(t * CH, CH)]], ss, add=True)

    def wait_s(s):
        ab, bb, cb, sa, sb, sc, ss = sets[s]
        pltpu.make_async_copy(cb, s_shared.at[irb.at[pl.ds(0, CH)]], ss).wait()

    issue(0, 0)
    wait_g(0)
    issue(1, 1)
    compute(0)
    scat_start(0, 0)

    def pair(i, _):
        t1 = 2 * i + 1
        wait_g(1)
        wait_s(0)
        issue(0, jnp.minimum(t1 + 1, CPW - 1))
        compute(1)
        scat_start(1, t1)
        t2 = t1 + 1
        wait_g(0)
        wait_s(1)
        issue(1, jnp.minimum(t2 + 1, CPW - 1))
        compute(0)
        scat_start(0, t2)
        return 0

    if CPW % 2 == 1:
        lax.fori_loop(0, (CPW - 1) // 2, pair, 0)
        wait_g(1)
        wait_s(0)
    else:
        lax.fori_loop(0, (CPW - 2) // 2, pair, 0)
        wait_g(1)
        compute(1)
        scat_start(1, CPW - 1)
        wait_s(0)
        wait_s(1)

    plsc.subcore_barrier()
    _writeout_shared(s_shared, out_hbm, cid, sid)

_edge_call = pl.kernel(
    _edge_body,
    out_type=jax.ShapeDtypeStruct((NC, N, H), f32),
    mesh=_MESH,
    scratch_types=[
        pltpu.VMEM_SHARED((N, H), f32),
        pltpu.VMEM((EPW,), jnp.int32),
        pltpu.VMEM((EPW,), jnp.int32),
        pltpu.VMEM((CH, H), f32),
        pltpu.VMEM((CH, H), f32),
        pltpu.VMEM((CH, H), f32),
        pltpu.VMEM((CH, H), f32),
        pltpu.VMEM((CH, H), f32),
        pltpu.VMEM((CH, H), f32),
    ] + [pltpu.SemaphoreType.DMA] * 8,
)


def _deg_body(row_hbm, out_hbm, d_shared, irb, vbuf, ss0, ss1):
    cid = lax.axis_index("c")
    sid = lax.axis_index("s")
    wid = sid * NC + cid

    _zero_fill(vbuf, CH, ED)
    _zero_shared(vbuf, d_shared, sid)
    _zero_fill(vbuf, CH, ED, value=1.0)
    pltpu.sync_copy(row_hbm.at[pl.ds(wid * EPW, EPW)], irb)
    plsc.subcore_barrier()

    def scat_start(sem, t):
        pltpu.async_copy(vbuf, d_shared.at[irb.at[pl.ds(t * CH, CH)]], sem,
                         add=True)

    def wait_s(sem):
        pltpu.make_async_copy(vbuf, d_shared.at[irb.at[pl.ds(0, CH)]], sem).wait()

    scat_start(ss0, 0)
    scat_start(ss1, 1)

    def chunk(i, _):
        wait_s(ss0)
        scat_start(ss0, 2 * i + 2)
        wait_s(ss1)
        scat_start(ss1, 2 * i + 3)
        return 0

    if CPW % 2 == 1:
        lax.fori_loop(0, (CPW - 3) // 2, chunk, 0)
        wait_s(ss0)
        scat_start(ss0, CPW - 1)
        wait_s(ss1)
        wait_s(ss0)
    else:
        lax.fori_loop(0, (CPW - 2) // 2, chunk, 0)
        wait_s(ss0)
        wait_s(ss1)
    plsc.subcore_barrier()
    _writeout_shared(d_shared, out_hbm, cid, sid)


_deg_call = pl.kernel(
    _deg_body,
    out_type=jax.ShapeDtypeStruct((NC, N, ED), f32),
    mesh=_MESH,
    scratch_types=[
        pltpu.VMEM_SHARED((N, ED), f32),
        pltpu.VMEM((EPW,), jnp.int32),
        pltpu.VMEM((CH, ED), f32),
        pltpu.SemaphoreType.DMA,
        pltpu.SemaphoreType.DMA,
    ],
)


def _full(shape):
    return pl.BlockSpec(shape, lambda i: (0,) * len(shape))


def _emb_ab_body(x_ref, embWT, embb, w1aT, w1bT, h_ref, a_ref, b_ref):
    h = jnp.dot(x_ref[...], embWT[...], preferred_element_type=f32) + embb[...]
    h_ref[...] = h
    a_ref[...] = jnp.dot(h, w1aT[...], preferred_element_type=f32)
    b_ref[...] = jnp.dot(h, w1bT[...], preferred_element_type=f32)


_emb_ab_call = pl.pallas_call(
    _emb_ab_body,
    grid=(NBLK,),
    in_specs=[
        pl.BlockSpec((BLK, D), lambda i: (i, 0)),
        _full((D, H)), _full((1, H)), _full((H, H)), _full((H, H)),
    ],
    out_specs=[pl.BlockSpec((BLK, H), lambda i: (i, 0))] * 3,
    out_shape=[jax.ShapeDtypeStruct((N, H), f32)] * 3,
)


def _edgec_body(ea_ref, w0, b0, c_ref):
    ea = ea_ref[...]
    c_ref[...] = jnp.dot(ea, w0[...], preferred_element_type=f32) + b0[...]


_edgec_call = pl.pallas_call(
    _edgec_body,
    grid=(NEBLK,),
    in_specs=[
        pl.BlockSpec((EBLK, ED), lambda i: (i, 0)),
        _full((ED, H)), _full((1, H)),
    ],
    out_specs=pl.BlockSpec((EBLK, H), lambda i: (i, 0)),
    out_shape=jax.ShapeDtypeStruct((E, H), f32),
)


def _gru_core(s2, deg2, h, w2T, b2, wihT, whhT, bih, bhh):
    s = s2[0] + s2[1]
    degc = deg2[0, :, :1] + deg2[1, :, :1]
    agg = jnp.dot(s, w2T[...], preferred_element_type=f32) + degc * b2[...]
    gi = jnp.dot(agg, wihT[...], preferred_element_type=f32) + bih[...]
    gh = jnp.dot(h, whhT[...], preferred_element_type=f32) + bhh[...]
    r = jax.nn.sigmoid(gi[:, :H] + gh[:, :H])
    z = jax.nn.sigmoid(gi[:, H:2 * H] + gh[:, H:2 * H])
    n = jnp.tanh(gi[:, 2 * H:] + r * gh[:, 2 * H:])
    return (1.0 - z) * n + z * h


def _gru_ab_body(s2, deg2, h_ref, w2T, b2, wihT, whhT, bih, bhh, w1aT, w1bT,
                 h1_ref, a_ref, b_ref):
    hn = _gru_core(s2, deg2, h_ref[...], w2T, b2, wihT, whhT, bih, bhh)
    h1_ref[...] = hn
    a_ref[...] = jnp.dot(hn, w1aT[...], preferred_element_type=f32)
    b_ref[...] = jnp.dot(hn, w1bT[...], preferred_element_type=f32)


_gru_ab_call = pl.pallas_call(
    _gru_ab_body,
    grid=(NBLK,),
    in_specs=[
        pl.BlockSpec((NC, BLK, H), lambda i: (0, i, 0)),
        pl.BlockSpec((NC, BLK, ED), lambda i: (0, i, 0)),
        pl.BlockSpec((BLK, H), lambda i: (i, 0)),
        _full((H, H)), _full((1, H)), _full((H, 3 * H)), _full((H, 3 * H)),
        _full((1, 3 * H)), _full((1, 3 * H)), _full((H, H)), _full((H, H)),
    ],
    out_specs=[pl.BlockSpec((BLK, H), lambda i: (i, 0))] * 3,
    out_shape=[jax.ShapeDtypeStruct((N, H), f32)] * 3,
)


def _gru_pool_body(s2, deg2, h_ref, w2T, b2, wihT, whhT, bih, bhh, batch_ref,
                   rw1T, rb1, rw2T, rb2, out_ref, sums, counts):
    i = pl.program_id(0)
    hn = _gru_core(s2, deg2, h_ref[...], w2T, b2, wihT, whhT, bih, bhh)
    bvec = batch_ref[0, 0, :]
    gids = lax.broadcasted_iota(jnp.int32, (G, BLK), 0)
    one_hot = (bvec[None, :] == gids).astype(f32)
    part = jnp.dot(one_hot, hn, preferred_element_type=f32)
    cnt = jnp.broadcast_to(jnp.sum(one_hot, axis=1, keepdims=True), (G, H))

    @pl.when(i == 0)
    def _():
        sums[...] = part
        counts[...] = cnt

    @pl.when(i > 0)
    def _():
        sums[...] += part
        counts[...] += cnt

    @pl.when(i == NBLK - 1)
    def _():
        pooled = sums[...] / jnp.maximum(counts[...], 1.0)
        t = jnp.maximum(
            jnp.dot(pooled, rw1T[...], preferred_element_type=f32) + rb1[...], 0.0
        )
        o = jnp.dot(t, rw2T[...], preferred_element_type=f32) + rb2[...]
        out_ref[...] = jax.nn.sigmoid(o)


_gru_pool_call = pl.pallas_call(
    _gru_pool_body,
    grid=(NBLK,),
    in_specs=[
        pl.BlockSpec((NC, BLK, H), lambda i: (0, i, 0)),
        pl.BlockSpec((NC, BLK, ED), lambda i: (0, i, 0)),
        pl.BlockSpec((BLK, H), lambda i: (i, 0)),
        _full((H, H)), _full((1, H)), _full((H, 3 * H)), _full((H, 3 * H)),
        _full((1, 3 * H)), _full((1, 3 * H)),
        pl.BlockSpec((1, 1, BLK), lambda i: (i, 0, 0)),
        _full((H, H)), _full((1, H)), _full((H, 1)), _full((1, 1)),
    ],
    out_specs=pl.BlockSpec((G, 1), lambda i: (0, 0)),
    out_shape=jax.ShapeDtypeStruct((G, 1), f32),
    scratch_shapes=[pltpu.VMEM((G, H), f32), pltpu.VMEM((G, H), f32)],
)


def kernel(x, edge_index, edge_attr, batch, emb_W, emb_b,
           l0_mW1, l0_mb1, l0_mW2, l0_mb2, l0_Wih, l0_Whh, l0_bih, l0_bhh,
           l1_mW1, l1_mb1, l1_mW2, l1_mb2, l1_Wih, l1_Whh, l1_bih, l1_bhh,
           r_W1, r_b1, r_W2, r_b2):
    row = edge_index[0]
    col = edge_index[1]
    batch3 = batch.astype(jnp.int32).reshape(NBLK, 1, BLK)

    def r1(v):
        return v.reshape(1, -1)

    w = {}
    for l, (mW1, mb1, mW2, mb2, Wih, Whh, bih, bhh) in enumerate(
        [(l0_mW1, l0_mb1, l0_mW2, l0_mb2, l0_Wih, l0_Whh, l0_bih, l0_bhh),
         (l1_mW1, l1_mb1, l1_mW2, l1_mb2, l1_Wih, l1_Whh, l1_bih, l1_bhh)]
    ):
        w[l] = dict(
            w1aT=mW1[:, :H].T, w1bT=mW1[:, H:2 * H].T, w1cT=mW1[:, 2 * H:].T,
            mb1=r1(mb1), w2T=mW2.T, mb2=r1(mb2), wihT=Wih.T, whhT=Whh.T,
            bih=r1(bih), bhh=r1(bhh),
        )

    c0 = _edgec_call(edge_attr, w[0]["w1cT"], w[0]["mb1"])
    c1 = _edgec_call(edge_attr, w[1]["w1cT"], w[1]["mb1"])
    deg2 = _deg_call(row)
    h0, a0, b0 = _emb_ab_call(x, emb_W.T, r1(emb_b), w[0]["w1aT"], w[0]["w1bT"])
    s0 = _edge_call(a0, b0, c0, row, col)
    h1, a1, b1 = _gru_ab_call(
        s0, deg2, h0, w[0]["w2T"], w[0]["mb2"], w[0]["wihT"], w[0]["whhT"],
        w[0]["bih"], w[0]["bhh"], w[1]["w1aT"], w[1]["w1bT"]
    )
    s1 = _edge_call(a1, b1, c1, row, col)
    out = _gru_pool_call(
        s1, deg2, h1, w[1]["w2T"], w[1]["mb2"], w[1]["wihT"], w[1]["whhT"],
        w[1]["bih"], w[1]["bhh"], batch3, r_W1.T, r1(r_b1), r_W2.T,
        r_b2.reshape(1, 1)
    )
    return out


# speedup vs baseline: 1.2067x; 1.1734x over previous
import functools

import jax
import jax.numpy as jnp
import numpy as np
from jax import lax
from jax.experimental import pallas as pl
from jax.experimental.pallas import tpu as pltpu
from jax.experimental.pallas import tpu_sc as plsc

N = 10000
E = 320000
D = 128
ED = 16
H = 128
G = 64

NC = 2
NS = 16
NW = NC * NS
CH = 40
NCHUNK = E // CH
EPW = E // NW
CPW = EPW // CH
RPT = 624
TAIL = N - NS * RPT

BLK = 1000
NBLK = N // BLK
EBLK = 4000
NEBLK = E // EBLK

f32 = jnp.float32

_MESH = plsc.VectorSubcoreMesh(
    core_axis_name="c", subcore_axis_name="s", num_cores=NC, num_subcores=NS
)


def _zero_fill(buf, rows, cols, value=0.0):
    def body(r, _):
        for j in range(cols // 16):
            buf[r, pl.ds(j * 16, 16)] = jnp.full((16,), value, f32)
        return 0

    lax.fori_loop(0, rows, body, 0)


def _zero_shared(vbuf, shared, sid):
    base = sid * RPT
    for k in range(RPT // CH):
        pltpu.sync_copy(vbuf, shared.at[pl.ds(base + k * CH, CH)])
    rem = RPT % CH
    if rem:
        pltpu.sync_copy(
            vbuf.at[pl.ds(0, rem)], shared.at[pl.ds(base + (RPT // CH) * CH, rem)]
        )

    @pl.when(sid == NS - 1)
    def _():
        pltpu.sync_copy(vbuf.at[pl.ds(0, TAIL)], shared.at[pl.ds(NS * RPT, TAIL)])


def _writeout_shared(shared, out_hbm, cid, sid):
    pltpu.sync_copy(
        shared.at[pl.ds(sid * RPT, RPT)], out_hbm.at[cid, pl.ds(sid * RPT, RPT)]
    )

    @pl.when(sid == NS - 1)
    def _():
        pltpu.sync_copy(
            shared.at[pl.ds(NS * RPT, TAIL)], out_hbm.at[cid, pl.ds(NS * RPT, TAIL)]
        )


def _edge_body(a_hbm, b_hbm, c_hbm, row_hbm, col_hbm, out_hbm,
               s_shared, irb, icb, a0, b0, c0, a1, b1, c1,
               sa0, sb0, sc0, sa1, sb1, sc1, ss0, ss1):
    sets = [(a0, b0, c0, sa0, sb0, sc0, ss0), (a1, b1, c1, sa1, sb1, sc1, ss1)]
    cid = lax.axis_index("c")
    sid = lax.axis_index("s")
    wid = sid * NC + cid

    _zero_fill(c0, CH, H)
    _zero_shared(c0, s_shared, sid)
    pltpu.sync_copy(row_hbm.at[pl.ds(wid * EPW, EPW)], irb)
    pltpu.sync_copy(col_hbm.at[pl.ds(wid * EPW, EPW)], icb)
    plsc.subcore_barrier()

    def issue_ab(s, t):
        ab, bb, cb, sa, sb, sc, ss = sets[s]
        pltpu.async_copy(a_hbm.at[irb.at[pl.ds(t * CH, CH)]], ab, sa)
        pltpu.async_copy(b_hbm.at[icb.at[pl.ds(t * CH, CH)]], bb, sb)

    def issue_c(s, t):
        ab, bb, cb, sa, sb, sc, ss = sets[s]
        pltpu.async_copy(c_hbm.at[pl.ds(wid * EPW + t * CH, CH)], cb, sc)

    def issue(s, t):
        issue_ab(s, t)
        issue_c(s, t)

    def wait_g(s):
        ab, bb, cb, sa, sb, sc, ss = sets[s]
        pltpu.make_async_copy(a_hbm.at[irb.at[pl.ds(0, CH)]], ab, sa).wait()
        pltpu.make_async_copy(b_hbm.at[icb.at[pl.ds(0, CH)]], bb, sb).wait()
        pltpu.make_async_copy(c_hbm.at[pl.ds(0, CH)], cb, sc).wait()

    def compute(s):
        ab, bb, cb, sa, sb, sc, ss = sets[s]

        def crow(r, _):
            for j in range(H // 16):
                sl = pl.ds(j * 16, 16)
                v = ab[r, sl] + bb[r, sl] + cb[r, sl]
                cb[r, sl] = jnp.maximum(v, 0.0)
            return 0

        lax.fori_loop(0, CH, crow, 0)

    def scat_start(s, t):
        ab, bb, cb, sa, sb, sc, ss = sets[s]
        pltpu.async_copy(cb, s_shared.at[irb.at[pl.ds(t * CH, CH)]], ss, add=True)

    def wait_s(s):
        ab, bb, cb, sa, sb, sc, ss = sets[s]
        pltpu.make_async_copy(cb, s_shared.at[irb.at[pl.ds(0, CH)]], ss).wait()

    issue(0, 0)
    wait_g(0)
    issue(1, 1)
    compute(0)
    scat_start(0, 0)

    def pair(i, _):
        t1 = 2 * i + 1
        t2 = jnp.minimum(t1 + 1, CPW - 1)
        t3 = jnp.minimum(t1 + 2, CPW - 1)
        issue_ab(0, t2)
        wait_g(1)
        wait_s(0)
        issue_c(0, t2)
        compute(1)
        scat_start(1, t1)
        issue_ab(1, t3)
        wait_g(0)
        wait_s(1)
        issue_c(1, t3)
        compute(0)
        scat_start(0, t1 + 1)
        return 0

    if CPW % 2 == 1:
        lax.fori_loop(0, (CPW - 1) // 2, pair, 0)
        wait_g(1)
        wait_s(0)
    else:
        lax.fori_loop(0, (CPW - 2) // 2, pair, 0)
        wait_g(1)
        compute(1)
        scat_start(1, CPW - 1)
        wait_s(0)
        wait_s(1)

    plsc.subcore_barrier()
    _writeout_shared(s_shared, out_hbm, cid, sid)

_edge_call = pl.kernel(
    _edge_body,
    out_type=jax.ShapeDtypeStruct((NC, N, H), f32),
    mesh=_MESH,
    scratch_types=[
        pltpu.VMEM_SHARED((N, H), f32),
        pltpu.VMEM((EPW,), jnp.int32),
        pltpu.VMEM((EPW,), jnp.int32),
        pltpu.VMEM((CH, H), f32),
        pltpu.VMEM((CH, H), f32),
        pltpu.VMEM((CH, H), f32),
        pltpu.VMEM((CH, H), f32),
        pltpu.VMEM((CH, H), f32),
        pltpu.VMEM((CH, H), f32),
    ] + [pltpu.SemaphoreType.DMA] * 8,
)


def _deg_body(row_hbm, out_hbm, d_shared, irb, vbuf, ss0, ss1):
    cid = lax.axis_index("c")
    sid = lax.axis_index("s")
    wid = sid * NC + cid

    _zero_fill(vbuf, CH, ED)
    _zero_shared(vbuf, d_shared, sid)
    _zero_fill(vbuf, CH, ED, value=1.0)
    pltpu.sync_copy(row_hbm.at[pl.ds(wid * EPW, EPW)], irb)
    plsc.subcore_barrier()

    def scat_start(sem, t):
        pltpu.async_copy(vbuf, d_shared.at[irb.at[pl.ds(t * CH, CH)]], sem,
                         add=True)

    def wait_s(sem):
        pltpu.make_async_copy(vbuf, d_shared.at[irb.at[pl.ds(0, CH)]], sem).wait()

    scat_start(ss0, 0)
    scat_start(ss1, 1)

    def chunk(i, _):
        wait_s(ss0)
        scat_start(ss0, 2 * i + 2)
        wait_s(ss1)
        scat_start(ss1, 2 * i + 3)
        return 0

    if CPW % 2 == 1:
        lax.fori_loop(0, (CPW - 3) // 2, chunk, 0)
        wait_s(ss0)
        scat_start(ss0, CPW - 1)
        wait_s(ss1)
        wait_s(ss0)
    else:
        lax.fori_loop(0, (CPW - 2) // 2, chunk, 0)
        wait_s(ss0)
        wait_s(ss1)
    plsc.subcore_barrier()
    _writeout_shared(d_shared, out_hbm, cid, sid)


_deg_call = pl.kernel(
    _deg_body,
    out_type=jax.ShapeDtypeStruct((NC, N, ED), f32),
    mesh=_MESH,
    scratch_types=[
        pltpu.VMEM_SHARED((N, ED), f32),
        pltpu.VMEM((EPW,), jnp.int32),
        pltpu.VMEM((CH, ED), f32),
        pltpu.SemaphoreType.DMA,
        pltpu.SemaphoreType.DMA,
    ],
)


def _full(shape):
    return pl.BlockSpec(shape, lambda i: (0,) * len(shape))


def _emb_ab_body(x_ref, embWT, embb, w1aT, w1bT, h_ref, a_ref, b_ref):
    h = jnp.dot(x_ref[...], embWT[...], preferred_element_type=f32) + embb[...]
    h_ref[...] = h
    a_ref[...] = jnp.dot(h, w1aT[...], preferred_element_type=f32)
    b_ref[...] = jnp.dot(h, w1bT[...], preferred_element_type=f32)


_emb_ab_call = pl.pallas_call(
    _emb_ab_body,
    grid=(NBLK,),
    in_specs=[
        pl.BlockSpec((BLK, D), lambda i: (i, 0)),
        _full((D, H)), _full((1, H)), _full((H, H)), _full((H, H)),
    ],
    out_specs=[pl.BlockSpec((BLK, H), lambda i: (i, 0))] * 3,
    out_shape=[jax.ShapeDtypeStruct((N, H), f32)] * 3,
)


def _edgec_body(ea_ref, w0, b0, w1, b1, c0_ref, c1_ref):
    ea = ea_ref[...]
    c0_ref[...] = jnp.dot(ea, w0[...], preferred_element_type=f32) + b0[...]
    c1_ref[...] = jnp.dot(ea, w1[...], preferred_element_type=f32) + b1[...]


_edgec_call = pl.pallas_call(
    _edgec_body,
    grid=(NEBLK,),
    in_specs=[
        pl.BlockSpec((EBLK, ED), lambda i: (i, 0)),
        _full((ED, H)), _full((1, H)), _full((ED, H)), _full((1, H)),
    ],
    out_specs=[pl.BlockSpec((EBLK, H), lambda i: (i, 0))] * 2,
    out_shape=[jax.ShapeDtypeStruct((E, H), f32)] * 2,
)


def _gru_core(s2, deg2, h, w2T, b2, wihT, whhT, bih, bhh):
    s = s2[0] + s2[1]
    degc = deg2[0, :, :1] + deg2[1, :, :1]
    agg = jnp.dot(s, w2T[...], preferred_element_type=f32) + degc * b2[...]
    gi = jnp.dot(agg, wihT[...], preferred_element_type=f32) + bih[...]
    gh = jnp.dot(h, whhT[...], preferred_element_type=f32) + bhh[...]
    r = jax.nn.sigmoid(gi[:, :H] + gh[:, :H])
    z = jax.nn.sigmoid(gi[:, H:2 * H] + gh[:, H:2 * H])
    n = jnp.tanh(gi[:, 2 * H:] + r * gh[:, 2 * H:])
    return (1.0 - z) * n + z * h


def _gru_ab_body(s2, deg2, h_ref, w2T, b2, wihT, whhT, bih, bhh, w1aT, w1bT,
                 h1_ref, a_ref, b_ref):
    hn = _gru_core(s2, deg2, h_ref[...], w2T, b2, wihT, whhT, bih, bhh)
    h1_ref[...] = hn
    a_ref[...] = jnp.dot(hn, w1aT[...], preferred_element_type=f32)
    b_ref[...] = jnp.dot(hn, w1bT[...], preferred_element_type=f32)


_gru_ab_call = pl.pallas_call(
    _gru_ab_body,
    grid=(NBLK,),
    in_specs=[
        pl.BlockSpec((NC, BLK, H), lambda i: (0, i, 0)),
        pl.BlockSpec((NC, BLK, ED), lambda i: (0, i, 0)),
        pl.BlockSpec((BLK, H), lambda i: (i, 0)),
        _full((H, H)), _full((1, H)), _full((H, 3 * H)), _full((H, 3 * H)),
        _full((1, 3 * H)), _full((1, 3 * H)), _full((H, H)), _full((H, H)),
    ],
    out_specs=[pl.BlockSpec((BLK, H), lambda i: (i, 0))] * 3,
    out_shape=[jax.ShapeDtypeStruct((N, H), f32)] * 3,
)


def _gru_pool_body(s2, deg2, h_ref, w2T, b2, wihT, whhT, bih, bhh, batch_ref,
                   rw1T, rb1, rw2T, rb2, out_ref, sums, counts):
    i = pl.program_id(0)
    hn = _gru_core(s2, deg2, h_ref[...], w2T, b2, wihT, whhT, bih, bhh)
    bvec = batch_ref[0, 0, :]
    gids = lax.broadcasted_iota(jnp.int32, (G, BLK), 0)
    one_hot = (bvec[None, :] == gids).astype(f32)
    part = jnp.dot(one_hot, hn, preferred_element_type=f32)
    cnt = jnp.broadcast_to(jnp.sum(one_hot, axis=1, keepdims=True), (G, H))

    @pl.when(i == 0)
    def _():
        sums[...] = part
        counts[...] = cnt

    @pl.when(i > 0)
    def _():
        sums[...] += part
        counts[...] += cnt

    @pl.when(i == NBLK - 1)
    def _():
        pooled = sums[...] / jnp.maximum(counts[...], 1.0)
        t = jnp.maximum(
            jnp.dot(pooled, rw1T[...], preferred_element_type=f32) + rb1[...], 0.0
        )
        o = jnp.dot(t, rw2T[...], preferred_element_type=f32) + rb2[...]
        out_ref[...] = jax.nn.sigmoid(o)


_gru_pool_call = pl.pallas_call(
    _gru_pool_body,
    grid=(NBLK,),
    in_specs=[
        pl.BlockSpec((NC, BLK, H), lambda i: (0, i, 0)),
        pl.BlockSpec((NC, BLK, ED), lambda i: (0, i, 0)),
        pl.BlockSpec((BLK, H), lambda i: (i, 0)),
        _full((H, H)), _full((1, H)), _full((H, 3 * H)), _full((H, 3 * H)),
        _full((1, 3 * H)), _full((1, 3 * H)),
        pl.BlockSpec((1, 1, BLK), lambda i: (i, 0, 0)),
        _full((H, H)), _full((1, H)), _full((H, 1)), _full((1, 1)),
    ],
    out_specs=pl.BlockSpec((G, 1), lambda i: (0, 0)),
    out_shape=jax.ShapeDtypeStruct((G, 1), f32),
    scratch_shapes=[pltpu.VMEM((G, H), f32), pltpu.VMEM((G, H), f32)],
)


def kernel(x, edge_index, edge_attr, batch, emb_W, emb_b,
           l0_mW1, l0_mb1, l0_mW2, l0_mb2, l0_Wih, l0_Whh, l0_bih, l0_bhh,
           l1_mW1, l1_mb1, l1_mW2, l1_mb2, l1_Wih, l1_Whh, l1_bih, l1_bhh,
           r_W1, r_b1, r_W2, r_b2):
    row = edge_index[0]
    col = edge_index[1]
    batch3 = batch.astype(jnp.int32).reshape(NBLK, 1, BLK)

    def r1(v):
        return v.reshape(1, -1)

    w = {}
    for l, (mW1, mb1, mW2, mb2, Wih, Whh, bih, bhh) in enumerate(
        [(l0_mW1, l0_mb1, l0_mW2, l0_mb2, l0_Wih, l0_Whh, l0_bih, l0_bhh),
         (l1_mW1, l1_mb1, l1_mW2, l1_mb2, l1_Wih, l1_Whh, l1_bih, l1_bhh)]
    ):
        w[l] = dict(
            w1aT=mW1[:, :H].T, w1bT=mW1[:, H:2 * H].T, w1cT=mW1[:, 2 * H:].T,
            mb1=r1(mb1), w2T=mW2.T, mb2=r1(mb2), wihT=Wih.T, whhT=Whh.T,
            bih=r1(bih), bhh=r1(bhh),
        )

    c0, c1 = _edgec_call(edge_attr, w[0]["w1cT"], w[0]["mb1"],
                         w[1]["w1cT"], w[1]["mb1"])
    deg2 = _deg_call(row)
    h0, a0, b0 = _emb_ab_call(x, emb_W.T, r1(emb_b), w[0]["w1aT"], w[0]["w1bT"])
    s0 = _edge_call(a0, b0, c0, row, col)
    h1, a1, b1 = _gru_ab_call(
        s0, deg2, h0, w[0]["w2T"], w[0]["mb2"], w[0]["wihT"], w[0]["whhT"],
        w[0]["bih"], w[0]["bhh"], w[1]["w1aT"], w[1]["w1bT"]
    )
    s1 = _edge_call(a1, b1, c1, row, col)
    out = _gru_pool_call(
        s1, deg2, h1, w[1]["w2T"], w[1]["mb2"], w[1]["wihT"], w[1]["whhT"],
        w[1]["bih"], w[1]["bhh"], batch3, r_W1.T, r1(r_b1), r_W2.T,
        r_b2.reshape(1, 1)
    )
    return out
